# baseline (speedup 1.0000x reference)
import math

import jax
import jax.numpy as jnp
from jax.experimental import pallas as pl
from jax.experimental.pallas import tpu as pltpu

_VMEM = 48 * 1024 * 1024
_EPS = 1e-5


def _gelu(x):
    c = math.sqrt(2.0 / math.pi)
    return 0.5 * x * (1.0 + jnp.tanh(c * (x + 0.044715 * x * x * x)))


def _ln_rows(x):
    mu = jnp.mean(x, axis=0, keepdims=True)
    var = jnp.mean((x - mu) ** 2, axis=0, keepdims=True)
    return (x - mu) * jax.lax.rsqrt(var + _EPS)


def _ln2_rows(x):
    return _ln_rows(_ln_rows(x))


def _init_grouped(key, ksize, cin, cout, groups):
    kt, kh, kw = ksize
    kkey, bkey = jax.random.split(key)
    cpg_in = cin // groups
    cpg_out = cout // groups
    scale = 1.0 / math.sqrt(kt * kh * kw * cpg_in)
    w = jax.random.normal(kkey, (kt, kh, kw, cin, cout), jnp.float32) * scale
    gi = jnp.arange(cin) // cpg_in
    go = jnp.arange(cout) // cpg_out
    w = w * (gi[:, None] == go[None, :]).astype(jnp.float32)
    b = jax.random.normal(bkey, (cout,), jnp.float32) * 0.02
    return w, b


def _init_dw(key, k, c):
    kkey, bkey = jax.random.split(key)
    w = jax.random.normal(kkey, (k, k, c), jnp.float32) * (1.0 / math.sqrt(k * k))
    b = jax.random.normal(bkey, (c,), jnp.float32) * 0.02
    return w, b


def _c1_body(a_ref, w_ref, b_ref, o_ref):
    acc = jnp.dot(w_ref[...], a_ref[0], preferred_element_type=jnp.float32)
    acc = _ln2_rows(_gelu(acc + b_ref[...]))
    o_ref[0] = acc.astype(o_ref.dtype)


def _c1_call(a, w_t, b_col, mt):
    B, K, M = a.shape
    C = w_t.shape[0]
    return pl.pallas_call(
        _c1_body,
        out_shape=jax.ShapeDtypeStruct((B, C, M), jnp.bfloat16),
        grid=(B, M // mt),
        in_specs=[pl.BlockSpec((1, K, mt), lambda b, i: (b, 0, i)),
                  pl.BlockSpec((C, K), lambda b, i: (0, 0)),
                  pl.BlockSpec((C, 1), lambda b, i: (0, 0))],
        out_specs=pl.BlockSpec((1, C, mt), lambda b, i: (b, 0, i)),
        compiler_params=pltpu.CompilerParams(
            dimension_semantics=("parallel", "parallel"),
            vmem_limit_bytes=_VMEM),
    )(a, w_t, b_col)


def _c1b_body(a_ref, w_ref, b_ref, o1_ref, o2_ref, o3_ref):
    acc = jnp.dot(w_ref[...], a_ref[0], preferred_element_type=jnp.float32)
    acc = _gelu(acc + b_ref[...])
    o1_ref[0] = _ln2_rows(acc[0:16]).astype(o1_ref.dtype)
    o2_ref[0] = _ln2_rows(acc[16:32]).astype(o2_ref.dtype)
    o3_ref[0] = _ln2_rows(acc[32:64]).astype(o3_ref.dtype)


def _c1b_call(a, w_t, b_col, mt):
    B, K, M = a.shape
    outs = [jax.ShapeDtypeStruct((B, 16, M), jnp.bfloat16),
            jax.ShapeDtypeStruct((B, 16, M), jnp.bfloat16),
            jax.ShapeDtypeStruct((B, 32, M), jnp.bfloat16)]
    return pl.pallas_call(
        _c1b_body,
        out_shape=outs,
        grid=(B, M // mt),
        in_specs=[pl.BlockSpec((1, K, mt), lambda b, i: (b, 0, i)),
                  pl.BlockSpec((64, K), lambda b, i: (0, 0)),
                  pl.BlockSpec((64, 1), lambda b, i: (0, 0))],
        out_specs=[pl.BlockSpec((1, 16, mt), lambda b, i: (b, 0, i)),
                   pl.BlockSpec((1, 16, mt), lambda b, i: (b, 0, i)),
                   pl.BlockSpec((1, 32, mt), lambda b, i: (b, 0, i))],
        compiler_params=pltpu.CompilerParams(
            dimension_semantics=("parallel", "parallel"),
            vmem_limit_bytes=_VMEM),
    )(a, w_t, b_col)


def _c2_weights(w2, C, C2):
    w00 = w2[0, :2, :2].reshape(4 * C, C2).T
    w01a = w2[0, 0, 2].T
    w01b = w2[0, 1, 2].T
    w10 = w2[0, 2, :2].reshape(2 * C, C2).T
    w11 = w2[0, 2, 2].T
    cat = jnp.concatenate([w00, w01a, w01b, w10, w11], axis=1)
    return cat.astype(jnp.bfloat16)


def _make_c2_body(C, X, YX):
    def body(h_ref, w_ref, b_ref, o_ref):
        h = h_ref[0]
        FL = 4 * YX
        tt = h.shape[1] // FL
        P = jnp.concatenate(
            [jnp.concatenate([h[:, f * FL + g * YX:f * FL + (g + 1) * YX]
                              for f in range(tt)], axis=1)
             for g in range(4)], axis=0)
        L = P.shape[1]

        def shift(v, s):
            return jnp.concatenate([v[:, s:], v[:, :s]], axis=1)

        lane = jax.lax.broadcasted_iota(jnp.int32, (1, L), 1)
        mx = (lane % X) < (X - 1)
        my = (lane % YX) < (YX - X)
        mxy = mx & my

        zero = jnp.zeros((), jnp.bfloat16)
        s1a = jnp.where(mx, shift(P[0:C], 1), zero)
        s1b = jnp.where(mx, shift(P[2 * C:3 * C], 1), zero)
        sx = jnp.where(my, shift(P[0:2 * C], X), zero)
        sx1 = jnp.where(mxy, shift(P[0:C], X + 1), zero)
        a = jnp.concatenate([P, s1a, s1b, sx, sx1], axis=0)

        acc = jnp.dot(w_ref[...], a, preferred_element_type=jnp.float32)
        acc = _ln_rows(_gelu(acc + b_ref[...]))
        o_ref[0] = acc.astype(o_ref.dtype)

    return body


def _c2_call(h, wcat, b_col, mt, X, YX):
    B, C, Min = h.shape
    M = Min // 4
    C2 = wcat.shape[0]
    return pl.pallas_call(
        _make_c2_body(C, X, YX),
        out_shape=jax.ShapeDtypeStruct((B, C2, M), jnp.bfloat16),
        grid=(B, M // mt),
        in_specs=[pl.BlockSpec((1, C, 4 * mt), lambda b, i: (b, 0, i)),
                  pl.BlockSpec((C2, 9 * C), lambda b, i: (0, 0)),
                  pl.BlockSpec((C2, 1), lambda b, i: (0, 0))],
        out_specs=pl.BlockSpec((1, C2, mt), lambda b, i: (b, 0, i)),
        compiler_params=pltpu.CompilerParams(
            dimension_semantics=("parallel", "parallel"),
            vmem_limit_bytes=_VMEM),
    )(h, wcat, b_col)


def _make_res_body(Wd, HW):
    taps = [(dy, dx) for dy in (-1, 0, 1) for dx in (-1, 0, 1)]

    def body(h_ref, w_ref, b_ref, pos_ref, o_ref):
        x = h_ref[0].astype(jnp.float32)
        C, L = x.shape
        lane = jax.lax.broadcasted_iota(jnp.int32, (1, L), 1)
        py = (lane % HW) // Wd
        px = lane % Wd

        masks = []
        for dy, dx in taps:
            ok = jnp.ones((1, L), jnp.bool_)
            if dy != 0:
                ok = ok & ((py + dy >= 0) & (py + dy < HW // Wd))
            if dx != 0:
                ok = ok & ((px + dx >= 0) & (px + dx < Wd))
            masks.append(ok)

        def conv(v, wcol0):
            acc = jnp.zeros_like(v)
            for t, (dy, dx) in enumerate(taps):
                off = dy * Wd + dx
                sh = pltpu.roll(v, (-off) % L, 1) if off else v
                sh = jnp.where(masks[t], sh, 0.0)
                acc = acc + sh * w_ref[:, wcol0 + t:wcol0 + t + 1]
            return acc

        def resblock(v, j):
            a1 = _ln_rows(_gelu(conv(v, 18 * j) + b_ref[:, 2 * j:2 * j + 1]))
            a2 = _ln_rows(_gelu(conv(a1, 18 * j + 9)
                                + b_ref[:, 2 * j + 1:2 * j + 2]))
            return a2 + v

        y = _ln_rows(resblock(resblock(x, 0), 1))
        o_ref[0] = pos_ref[...] + y.astype(jnp.bfloat16).astype(jnp.float32)

    return body


def _res_call(h, wcols, bcols, pos, Wd, HW, lt):
    B, C, M = h.shape
    return pl.pallas_call(
        _make_res_body(Wd, HW),
        out_shape=jax.ShapeDtypeStruct((B, C, M), jnp.float32),
        grid=(B, M // lt),
        in_specs=[pl.BlockSpec((1, C, lt), lambda b, i: (b, 0, i)),
                  pl.BlockSpec((C, 36), lambda b, i: (0, 0)),
                  pl.BlockSpec((C, 4), lambda b, i: (0, 0)),
                  pl.BlockSpec((C, lt), lambda b, i: (0, i))],
        out_specs=pl.BlockSpec((1, C, lt), lambda b, i: (b, 0, i)),
        compiler_params=pltpu.CompilerParams(
            dimension_semantics=("parallel", "parallel"),
            vmem_limit_bytes=_VMEM),
    )(h, wcols, bcols, pos)


def _res_cols(w4, b4):
    return w4.reshape(4 * 9, -1).T, b4.T


def _wt(w, K, C):
    return w.reshape(K, C).T.astype(jnp.bfloat16)


def _bcol(b):
    return b.reshape(-1, 1).astype(jnp.float32)


def kernel(x, params_key_data):
    key = jax.random.wrap_key_data(params_key_data)
    ks = jax.random.split(key, 8)
    bkeys = jax.vmap(lambda k: jax.random.split(k, 8))(ks[:4])

    w1, b1 = _init_grouped(bkeys[0, 0], (1, 2, 2), 4, 16, 4)
    wp_2, bp_2 = jax.vmap(lambda k: _init_grouped(k, (2, 4, 4), 4, 16, 4))(
        jnp.stack([bkeys[1, 0], bkeys[3, 0]]))
    wp1, bp1, wp2, bp2 = wp_2[0], bp_2[0], wp_2[1], bp_2[1]
    w2c, b2c = _init_grouped(bkeys[2, 0], (2, 4, 4), 4, 32, 4)

    w2_3, b2_3 = jax.vmap(lambda k: _init_grouped(k, (1, 3, 3), 16, 32, 16))(
        jnp.stack([bkeys[0, 1], bkeys[1, 1], bkeys[3, 1]]))
    w2_1, b2_1 = w2_3[0], b2_3[0]
    w2_p1, b2_p1 = w2_3[1], b2_3[1]
    w2_p2, b2_p2 = w2_3[2], b2_3[2]
    w2_2, b2_2 = _init_grouped(bkeys[2, 1], (1, 3, 3), 32, 64, 32)

    dwk32 = jnp.concatenate([bkeys[0, 2:6], bkeys[1, 2:6], bkeys[3, 2:6]])
    dww32, dwb32 = jax.vmap(lambda k: _init_dw(k, 3, 32))(dwk32)
    dww64, dwb64 = jax.vmap(lambda k: _init_dw(k, 3, 64))(bkeys[2, 2:6])

    pos1 = jax.random.normal(ks[4], (32, 16, 32, 32), jnp.float32) * 0.02
    pos2 = jax.random.normal(ks[5], (64, 8, 16, 16), jnp.float32) * 0.02
    pp = jax.vmap(
        lambda k: jax.random.normal(k, (32, 8, 16, 16), jnp.float32))(ks[6:8])
    pp1, pp2 = pp[0] * 0.02, pp[1] * 0.02

    B = x.shape[0]
    xb = x.astype(jnp.bfloat16)

    a1 = xb.reshape(B, 4, 16, 32, 2, 2, 32, 2, 2)
    a1 = a1.transpose(0, 5, 8, 1, 2, 4, 7, 3, 6).reshape(B, 16, 16 * 64 * 64)
    a2 = xb.reshape(B, 4, 8, 2, 16, 2, 4, 16, 2, 4)
    a2 = a2.transpose(0, 3, 6, 9, 1, 2, 5, 8, 4, 7).reshape(B, 128, 8 * 32 * 32)

    h1 = _c1_call(a1, _wt(w1, 16, 16), _bcol(b1), mt=8192)

    wcat = jnp.concatenate(
        [_wt(wp1, 128, 16), _wt(wp2, 128, 16), _wt(w2c, 128, 32)], axis=0)
    bcat = jnp.concatenate([_bcol(bp1), _bcol(bp2), _bcol(b2c)], axis=0)
    hp1, hp2, h2 = _c1b_call(a2, wcat, bcat, mt=2048)

    g1 = _c2_call(h1, _c2_weights(w2_1, 16, 32), _bcol(b2_1),
                  mt=2048, X=32, YX=1024)
    gp1 = _c2_call(hp1, _c2_weights(w2_p1, 16, 32), _bcol(b2_p1),
                   mt=2048, X=16, YX=256)
    gp2 = _c2_call(hp2, _c2_weights(w2_p2, 16, 32), _bcol(b2_p2),
                   mt=2048, X=16, YX=256)
    g2 = _c2_call(h2, _c2_weights(w2_2, 32, 64), _bcol(b2_2),
                  mt=2048, X=16, YX=256)

    o1 = _res_call(g1, *_res_cols(dww32[0:4], dwb32[0:4]), pos1.reshape(32, -1),
                   Wd=32, HW=1024, lt=4096)
    op1 = _res_call(gp1, *_res_cols(dww32[4:8], dwb32[4:8]), pp1.reshape(32, -1),
                    Wd=16, HW=256, lt=2048)
    op2 = _res_call(gp2, *_res_cols(dww32[8:12], dwb32[8:12]),
                    pp2.reshape(32, -1), Wd=16, HW=256, lt=2048)
    o2 = _res_call(g2, *_res_cols(dww64, dwb64), pos2.reshape(64, -1),
                   Wd=16, HW=256, lt=2048)

    return (o1.reshape(B, 32, 16, 32, 32),
            op1.reshape(B, 32, 8, 16, 16),
            o2.reshape(B, 64, 8, 16, 16),
            op2.reshape(B, 32, 8, 16, 16))


# speedup vs baseline: 3.6061x; 3.6061x over previous
import math

import jax
import jax.numpy as jnp
from jax.experimental import pallas as pl
from jax.experimental.pallas import tpu as pltpu

_VMEM = 48 * 1024 * 1024
_EPS = 1e-5


def _gelu(x):
    c = math.sqrt(2.0 / math.pi)
    return 0.5 * x * (1.0 + jnp.tanh(c * (x + 0.044715 * x * x * x)))


def _ln_rows(x):
    mu = jnp.mean(x, axis=0, keepdims=True)
    var = jnp.mean((x - mu) ** 2, axis=0, keepdims=True)
    return (x - mu) * jax.lax.rsqrt(var + _EPS)


def _ln2_rows(x):
    return _ln_rows(_ln_rows(x))


def _init_grouped(key, ksize, cin, cout, groups):
    kt, kh, kw = ksize
    kkey, bkey = jax.random.split(key)
    cpg_in = cin // groups
    cpg_out = cout // groups
    scale = 1.0 / math.sqrt(kt * kh * kw * cpg_in)
    w = jax.random.normal(kkey, (kt, kh, kw, cin, cout), jnp.float32) * scale
    gi = jnp.arange(cin) // cpg_in
    go = jnp.arange(cout) // cpg_out
    w = w * (gi[:, None] == go[None, :]).astype(jnp.float32)
    b = jax.random.normal(bkey, (cout,), jnp.float32) * 0.02
    return w, b


def _init_dw(key, k, c):
    kkey, bkey = jax.random.split(key)
    w = jax.random.normal(kkey, (k, k, c), jnp.float32) * (1.0 / math.sqrt(k * k))
    b = jax.random.normal(bkey, (c,), jnp.float32) * 0.02
    return w, b


def _c1_body(a_ref, w_ref, b_ref, o_ref):
    acc = jnp.dot(w_ref[...], a_ref[0], preferred_element_type=jnp.float32)
    acc = _ln2_rows(_gelu(acc + b_ref[...]))
    o_ref[0] = acc.astype(o_ref.dtype)


def _c1_call(a, w_t, b_col, mt):
    B, K, M = a.shape
    C = w_t.shape[0]
    return pl.pallas_call(
        _c1_body,
        out_shape=jax.ShapeDtypeStruct((B, C, M), jnp.bfloat16),
        grid=(B, M // mt),
        in_specs=[pl.BlockSpec((1, K, mt), lambda b, i: (b, 0, i)),
                  pl.BlockSpec((C, K), lambda b, i: (0, 0)),
                  pl.BlockSpec((C, 1), lambda b, i: (0, 0))],
        out_specs=pl.BlockSpec((1, C, mt), lambda b, i: (b, 0, i)),
        compiler_params=pltpu.CompilerParams(
            dimension_semantics=("parallel", "parallel"),
            vmem_limit_bytes=_VMEM),
    )(a, w_t, b_col)


def _c1b_body(a_ref, w_ref, b_ref, o1_ref, o2_ref, o3_ref):
    acc = jnp.dot(w_ref[...], a_ref[0], preferred_element_type=jnp.float32)
    acc = _gelu(acc + b_ref[...])
    o1_ref[0] = _ln2_rows(acc[0:16]).astype(o1_ref.dtype)
    o2_ref[0] = _ln2_rows(acc[16:32]).astype(o2_ref.dtype)
    o3_ref[0] = _ln2_rows(acc[32:64]).astype(o3_ref.dtype)


def _c1b_call(a, w_t, b_col, mt):
    B, K, M = a.shape
    outs = [jax.ShapeDtypeStruct((B, 16, M), jnp.bfloat16),
            jax.ShapeDtypeStruct((B, 16, M), jnp.bfloat16),
            jax.ShapeDtypeStruct((B, 32, M), jnp.bfloat16)]
    return pl.pallas_call(
        _c1b_body,
        out_shape=outs,
        grid=(B, M // mt),
        in_specs=[pl.BlockSpec((1, K, mt), lambda b, i: (b, 0, i)),
                  pl.BlockSpec((64, K), lambda b, i: (0, 0)),
                  pl.BlockSpec((64, 1), lambda b, i: (0, 0))],
        out_specs=[pl.BlockSpec((1, 16, mt), lambda b, i: (b, 0, i)),
                   pl.BlockSpec((1, 16, mt), lambda b, i: (b, 0, i)),
                   pl.BlockSpec((1, 32, mt), lambda b, i: (b, 0, i))],
        compiler_params=pltpu.CompilerParams(
            dimension_semantics=("parallel", "parallel"),
            vmem_limit_bytes=_VMEM),
    )(a, w_t, b_col)


def _c2_weights(w2, C, C2):
    w00 = w2[0, :2, :2].reshape(4 * C, C2).T
    w01a = w2[0, 0, 2].T
    w01b = w2[0, 1, 2].T
    w10 = w2[0, 2, :2].reshape(2 * C, C2).T
    w11 = w2[0, 2, 2].T
    cat = jnp.concatenate([w00, w01a, w01b, w10, w11], axis=1)
    return cat.astype(jnp.bfloat16)


def _phase_split(h, C, T, H, W):
    B = h.shape[0]
    Y, X = H // 2, W // 2
    e = h.reshape(B, C, T, Y, 2, X, 2).transpose(0, 4, 6, 1, 2, 3, 5)
    return e.reshape(B, 4 * C, T * Y * X)


def _make_c2_body(C, X, YX):
    def body(p_ref, w_ref, b_ref, o_ref):
        P = p_ref[0]
        L = P.shape[1]

        def shift(v, s):
            return jnp.concatenate([v[:, s:], v[:, :s]], axis=1)

        lane = jax.lax.broadcasted_iota(jnp.int32, (1, L), 1)
        mx = (lane % X) < (X - 1)
        my = (lane % YX) < (YX - X)
        mxy = mx & my

        zero = jnp.zeros((), jnp.bfloat16)
        s1a = jnp.where(mx, shift(P[0:C], 1), zero)
        s1b = jnp.where(mx, shift(P[2 * C:3 * C], 1), zero)
        sx = jnp.where(my, shift(P[0:2 * C], X), zero)
        sx1 = jnp.where(mxy, shift(P[0:C], X + 1), zero)
        a = jnp.concatenate([P, s1a, s1b, sx, sx1], axis=0)

        acc = jnp.dot(w_ref[...], a, preferred_element_type=jnp.float32)
        acc = _ln_rows(_gelu(acc + b_ref[...]))
        o_ref[0] = acc.astype(o_ref.dtype)

    return body


def _c2_call(ph, wcat, b_col, mt, X, YX):
    B, C4, M = ph.shape
    C = C4 // 4
    C2 = wcat.shape[0]
    return pl.pallas_call(
        _make_c2_body(C, X, YX),
        out_shape=jax.ShapeDtypeStruct((B, C2, M), jnp.bfloat16),
        grid=(B, M // mt),
        in_specs=[pl.BlockSpec((1, C4, mt), lambda b, i: (b, 0, i)),
                  pl.BlockSpec((C2, 9 * C), lambda b, i: (0, 0)),
                  pl.BlockSpec((C2, 1), lambda b, i: (0, 0))],
        out_specs=pl.BlockSpec((1, C2, mt), lambda b, i: (b, 0, i)),
        compiler_params=pltpu.CompilerParams(
            dimension_semantics=("parallel", "parallel"),
            vmem_limit_bytes=_VMEM),
    )(ph, wcat, b_col)


def _make_res_body(Wd, HW):
    taps = [(dy, dx) for dy in (-1, 0, 1) for dx in (-1, 0, 1)]

    def body(h_ref, w_ref, b_ref, pos_ref, o_ref):
        x = h_ref[0].astype(jnp.float32)
        C, L = x.shape
        lane = jax.lax.broadcasted_iota(jnp.int32, (1, L), 1)
        py = (lane % HW) // Wd
        px = lane % Wd

        masks = []
        for dy, dx in taps:
            ok = jnp.ones((1, L), jnp.bool_)
            if dy != 0:
                ok = ok & ((py + dy >= 0) & (py + dy < HW // Wd))
            if dx != 0:
                ok = ok & ((px + dx >= 0) & (px + dx < Wd))
            masks.append(ok)

        def conv(v, wcol0):
            acc = jnp.zeros_like(v)
            for t, (dy, dx) in enumerate(taps):
                off = dy * Wd + dx
                sh = pltpu.roll(v, (-off) % L, 1) if off else v
                sh = jnp.where(masks[t], sh, 0.0)
                acc = acc + sh * w_ref[:, wcol0 + t:wcol0 + t + 1]
            return acc

        def resblock(v, j):
            a1 = _ln_rows(_gelu(conv(v, 18 * j) + b_ref[:, 2 * j:2 * j + 1]))
            a2 = _ln_rows(_gelu(conv(a1, 18 * j + 9)
                                + b_ref[:, 2 * j + 1:2 * j + 2]))
            return a2 + v

        y = _ln_rows(resblock(resblock(x, 0), 1))
        o_ref[0] = pos_ref[...] + y.astype(jnp.bfloat16).astype(jnp.float32)

    return body


def _res_call(h, wcols, bcols, pos, Wd, HW, lt):
    B, C, M = h.shape
    return pl.pallas_call(
        _make_res_body(Wd, HW),
        out_shape=jax.ShapeDtypeStruct((B, C, M), jnp.float32),
        grid=(B, M // lt),
        in_specs=[pl.BlockSpec((1, C, lt), lambda b, i: (b, 0, i)),
                  pl.BlockSpec((C, 36), lambda b, i: (0, 0)),
                  pl.BlockSpec((C, 4), lambda b, i: (0, 0)),
                  pl.BlockSpec((C, lt), lambda b, i: (0, i))],
        out_specs=pl.BlockSpec((1, C, lt), lambda b, i: (b, 0, i)),
        compiler_params=pltpu.CompilerParams(
            dimension_semantics=("parallel", "parallel"),
            vmem_limit_bytes=_VMEM),
    )(h, wcols, bcols, pos)


def _res_cols(w4, b4):
    return w4.reshape(4 * 9, -1).T, b4.T


def _wt(w, K, C):
    return w.reshape(K, C).T.astype(jnp.bfloat16)


def _bcol(b):
    return b.reshape(-1, 1).astype(jnp.float32)


def kernel(x, params_key_data):
    key = jax.random.wrap_key_data(params_key_data)
    ks = jax.random.split(key, 8)
    bkeys = jax.vmap(lambda k: jax.random.split(k, 8))(ks[:4])

    w1, b1 = _init_grouped(bkeys[0, 0], (1, 2, 2), 4, 16, 4)
    wp_2, bp_2 = jax.vmap(lambda k: _init_grouped(k, (2, 4, 4), 4, 16, 4))(
        jnp.stack([bkeys[1, 0], bkeys[3, 0]]))
    wp1, bp1, wp2, bp2 = wp_2[0], bp_2[0], wp_2[1], bp_2[1]
    w2c, b2c = _init_grouped(bkeys[2, 0], (2, 4, 4), 4, 32, 4)

    w2_3, b2_3 = jax.vmap(lambda k: _init_grouped(k, (1, 3, 3), 16, 32, 16))(
        jnp.stack([bkeys[0, 1], bkeys[1, 1], bkeys[3, 1]]))
    w2_1, b2_1 = w2_3[0], b2_3[0]
    w2_p1, b2_p1 = w2_3[1], b2_3[1]
    w2_p2, b2_p2 = w2_3[2], b2_3[2]
    w2_2, b2_2 = _init_grouped(bkeys[2, 1], (1, 3, 3), 32, 64, 32)

    dwk32 = jnp.concatenate([bkeys[0, 2:6], bkeys[1, 2:6], bkeys[3, 2:6]])
    dww32, dwb32 = jax.vmap(lambda k: _init_dw(k, 3, 32))(dwk32)
    dww64, dwb64 = jax.vmap(lambda k: _init_dw(k, 3, 64))(bkeys[2, 2:6])

    pos1 = jax.random.normal(ks[4], (32, 16, 32, 32), jnp.float32) * 0.02
    pos2 = jax.random.normal(ks[5], (64, 8, 16, 16), jnp.float32) * 0.02
    pp = jax.vmap(
        lambda k: jax.random.normal(k, (32, 8, 16, 16), jnp.float32))(ks[6:8])
    pp1, pp2 = pp[0] * 0.02, pp[1] * 0.02

    B = x.shape[0]
    xb = x.astype(jnp.bfloat16)

    a1 = xb.reshape(B, 4, 16, 64, 2, 64, 2).transpose(0, 4, 6, 1, 2, 3, 5)
    a1 = a1.reshape(B, 16, 16 * 64 * 64)
    a2 = xb.reshape(B, 4, 8, 2, 32, 4, 32, 4).transpose(0, 3, 5, 7, 1, 2, 4, 6)
    a2 = a2.reshape(B, 128, 8 * 32 * 32)

    h1 = _c1_call(a1, _wt(w1, 16, 16), _bcol(b1), mt=8192)

    wcat = jnp.concatenate(
        [_wt(wp1, 128, 16), _wt(wp2, 128, 16), _wt(w2c, 128, 32)], axis=0)
    bcat = jnp.concatenate([_bcol(bp1), _bcol(bp2), _bcol(b2c)], axis=0)
    hp1, hp2, h2 = _c1b_call(a2, wcat, bcat, mt=2048)

    g1 = _c2_call(_phase_split(h1, 16, 16, 64, 64),
                  _c2_weights(w2_1, 16, 32), _bcol(b2_1),
                  mt=2048, X=32, YX=1024)
    gp1 = _c2_call(_phase_split(hp1, 16, 8, 32, 32),
                   _c2_weights(w2_p1, 16, 32), _bcol(b2_p1),
                   mt=2048, X=16, YX=256)
    gp2 = _c2_call(_phase_split(hp2, 16, 8, 32, 32),
                   _c2_weights(w2_p2, 16, 32), _bcol(b2_p2),
                   mt=2048, X=16, YX=256)
    g2 = _c2_call(_phase_split(h2, 32, 8, 32, 32),
                  _c2_weights(w2_2, 32, 64), _bcol(b2_2),
                  mt=2048, X=16, YX=256)

    o1 = _res_call(g1, *_res_cols(dww32[0:4], dwb32[0:4]), pos1.reshape(32, -1),
                   Wd=32, HW=1024, lt=4096)
    op1 = _res_call(gp1, *_res_cols(dww32[4:8], dwb32[4:8]), pp1.reshape(32, -1),
                    Wd=16, HW=256, lt=2048)
    op2 = _res_call(gp2, *_res_cols(dww32[8:12], dwb32[8:12]),
                    pp2.reshape(32, -1), Wd=16, HW=256, lt=2048)
    o2 = _res_call(g2, *_res_cols(dww64, dwb64), pos2.reshape(64, -1),
                   Wd=16, HW=256, lt=2048)

    return (o1.reshape(B, 32, 16, 32, 32),
            op1.reshape(B, 32, 8, 16, 16),
            o2.reshape(B, 64, 8, 16, 16),
            op2.reshape(B, 32, 8, 16, 16))


# speedup vs baseline: 3.6956x; 1.0248x over previous
import math

import jax
import jax.numpy as jnp
from jax.experimental import pallas as pl
from jax.experimental.pallas import tpu as pltpu

_VMEM = 48 * 1024 * 1024
_EPS = 1e-5


def _gelu(x):
    c = math.sqrt(2.0 / math.pi)
    return 0.5 * x * (1.0 + jnp.tanh(c * (x + 0.044715 * x * x * x)))


def _ln_rows(x):
    mu = jnp.mean(x, axis=0, keepdims=True)
    var = jnp.mean((x - mu) ** 2, axis=0, keepdims=True)
    return (x - mu) * jax.lax.rsqrt(var + _EPS)


def _ln2_rows(x):
    return _ln_rows(_ln_rows(x))


def _init_grouped(key, ksize, cin, cout, groups):
    kt, kh, kw = ksize
    kkey, bkey = jax.random.split(key)
    cpg_in = cin // groups
    cpg_out = cout // groups
    scale = 1.0 / math.sqrt(kt * kh * kw * cpg_in)
    w = jax.random.normal(kkey, (kt, kh, kw, cin, cout), jnp.float32) * scale
    gi = jnp.arange(cin) // cpg_in
    go = jnp.arange(cout) // cpg_out
    w = w * (gi[:, None] == go[None, :]).astype(jnp.float32)
    b = jax.random.normal(bkey, (cout,), jnp.float32) * 0.02
    return w, b


def _init_dw(key, k, c):
    kkey, bkey = jax.random.split(key)
    w = jax.random.normal(kkey, (k, k, c), jnp.float32) * (1.0 / math.sqrt(k * k))
    b = jax.random.normal(bkey, (c,), jnp.float32) * 0.02
    return w, b


def _c1_body(a_ref, w_ref, b_ref, o_ref):
    acc = jnp.dot(w_ref[...], a_ref[0], preferred_element_type=jnp.float32)
    acc = _ln2_rows(_gelu(acc + b_ref[...]))
    o_ref[0] = acc.astype(o_ref.dtype)


def _c1_call(a, w_t, b_col, mt):
    B, K, M = a.shape
    C = w_t.shape[0]
    return pl.pallas_call(
        _c1_body,
        out_shape=jax.ShapeDtypeStruct((B, C, M), jnp.bfloat16),
        grid=(B, M // mt),
        in_specs=[pl.BlockSpec((1, K, mt), lambda b, i: (b, 0, i)),
                  pl.BlockSpec((C, K), lambda b, i: (0, 0)),
                  pl.BlockSpec((C, 1), lambda b, i: (0, 0))],
        out_specs=pl.BlockSpec((1, C, mt), lambda b, i: (b, 0, i)),
        compiler_params=pltpu.CompilerParams(
            dimension_semantics=("parallel", "parallel"),
            vmem_limit_bytes=_VMEM),
    )(a, w_t, b_col)


def _c1b_body(a_ref, w_ref, b_ref, o1_ref, o2_ref, o3_ref):
    acc = jnp.dot(w_ref[...], a_ref[0], preferred_element_type=jnp.float32)
    acc = _gelu(acc + b_ref[...])
    o1_ref[0] = _ln2_rows(acc[0:16]).astype(o1_ref.dtype)
    o2_ref[0] = _ln2_rows(acc[16:32]).astype(o2_ref.dtype)
    o3_ref[0] = _ln2_rows(acc[32:64]).astype(o3_ref.dtype)


def _c1b_call(a, w_t, b_col, mt):
    B, K, M = a.shape
    outs = [jax.ShapeDtypeStruct((B, 16, M), jnp.bfloat16),
            jax.ShapeDtypeStruct((B, 16, M), jnp.bfloat16),
            jax.ShapeDtypeStruct((B, 32, M), jnp.bfloat16)]
    return pl.pallas_call(
        _c1b_body,
        out_shape=outs,
        grid=(B, M // mt),
        in_specs=[pl.BlockSpec((1, K, mt), lambda b, i: (b, 0, i)),
                  pl.BlockSpec((64, K), lambda b, i: (0, 0)),
                  pl.BlockSpec((64, 1), lambda b, i: (0, 0))],
        out_specs=[pl.BlockSpec((1, 16, mt), lambda b, i: (b, 0, i)),
                   pl.BlockSpec((1, 16, mt), lambda b, i: (b, 0, i)),
                   pl.BlockSpec((1, 32, mt), lambda b, i: (b, 0, i))],
        compiler_params=pltpu.CompilerParams(
            dimension_semantics=("parallel", "parallel"),
            vmem_limit_bytes=_VMEM),
    )(a, w_t, b_col)


def _c2_weights(w2, C, C2):
    w00 = w2[0, :2, :2].reshape(4 * C, C2).T
    w01a = w2[0, 0, 2].T
    w01b = w2[0, 1, 2].T
    w10 = w2[0, 2, :2].reshape(2 * C, C2).T
    w11 = w2[0, 2, 2].T
    cat = jnp.concatenate([w00, w01a, w01b, w10, w11], axis=1)
    return cat.astype(jnp.bfloat16)


def _phase_split(h, C, T, H, W):
    B = h.shape[0]
    Y, X = H // 2, W // 2
    e = h.reshape(B, C, T, Y, 2, X, 2).transpose(0, 4, 6, 1, 2, 3, 5)
    return e.reshape(B, 4 * C, T * Y * X)


def _make_c2_body(C, X, YX):
    def body(p_ref, w_ref, b_ref, o_ref):
        P = p_ref[0]
        L = P.shape[1]

        def shift(v, s):
            return jnp.concatenate([v[:, s:], v[:, :s]], axis=1)

        lane = jax.lax.broadcasted_iota(jnp.int32, (1, L), 1)
        mx = (lane % X) < (X - 1)
        my = (lane % YX) < (YX - X)
        mxy = mx & my

        zero = jnp.zeros((), jnp.bfloat16)
        s1a = jnp.where(mx, shift(P[0:C], 1), zero)
        s1b = jnp.where(mx, shift(P[2 * C:3 * C], 1), zero)
        sx = jnp.where(my, shift(P[0:2 * C], X), zero)
        sx1 = jnp.where(mxy, shift(P[0:C], X + 1), zero)
        a = jnp.concatenate([P, s1a, s1b, sx, sx1], axis=0)

        acc = jnp.dot(w_ref[...], a, preferred_element_type=jnp.float32)
        acc = _ln_rows(_gelu(acc + b_ref[...]))
        o_ref[0] = acc.astype(o_ref.dtype)

    return body


def _c2_call(ph, wcat, b_col, mt, X, YX):
    B, C4, M = ph.shape
    C = C4 // 4
    C2 = wcat.shape[0]
    return pl.pallas_call(
        _make_c2_body(C, X, YX),
        out_shape=jax.ShapeDtypeStruct((B, C2, M), jnp.bfloat16),
        grid=(B, M // mt),
        in_specs=[pl.BlockSpec((1, C4, mt), lambda b, i: (b, 0, i)),
                  pl.BlockSpec((C2, 9 * C), lambda b, i: (0, 0)),
                  pl.BlockSpec((C2, 1), lambda b, i: (0, 0))],
        out_specs=pl.BlockSpec((1, C2, mt), lambda b, i: (b, 0, i)),
        compiler_params=pltpu.CompilerParams(
            dimension_semantics=("parallel", "parallel"),
            vmem_limit_bytes=_VMEM),
    )(ph, wcat, b_col)


def _make_res_body(Wd, HW):
    taps = [(dy, dx) for dy in (-1, 0, 1) for dx in (-1, 0, 1)]

    def body(h_ref, w_ref, b_ref, pos_ref, o_ref):
        x = h_ref[0].astype(jnp.float32)
        C, L = x.shape
        lane = jax.lax.broadcasted_iota(jnp.int32, (1, L), 1)
        py = (lane % HW) // Wd
        px = lane % Wd

        masks = []
        for dy, dx in taps:
            ok = jnp.ones((1, L), jnp.bool_)
            if dy != 0:
                ok = ok & ((py + dy >= 0) & (py + dy < HW // Wd))
            if dx != 0:
                ok = ok & ((px + dx >= 0) & (px + dx < Wd))
            masks.append(ok)

        def conv(v, wcol0):
            acc = jnp.zeros_like(v)
            for t, (dy, dx) in enumerate(taps):
                off = dy * Wd + dx
                sh = pltpu.roll(v, (-off) % L, 1) if off else v
                sh = jnp.where(masks[t], sh, 0.0)
                acc = acc + sh * w_ref[:, wcol0 + t:wcol0 + t + 1]
            return acc

        def resblock(v, j):
            a1 = _ln_rows(_gelu(conv(v, 18 * j) + b_ref[:, 2 * j:2 * j + 1]))
            a2 = _ln_rows(_gelu(conv(a1, 18 * j + 9)
                                + b_ref[:, 2 * j + 1:2 * j + 2]))
            return a2 + v

        y = _ln_rows(resblock(resblock(x, 0), 1))
        o_ref[0] = pos_ref[...] + y.astype(jnp.bfloat16).astype(jnp.float32)

    return body


def _res_call(h, wcols, bcols, pos, Wd, HW, lt):
    B, C, M = h.shape
    return pl.pallas_call(
        _make_res_body(Wd, HW),
        out_shape=jax.ShapeDtypeStruct((B, C, M), jnp.float32),
        grid=(B, M // lt),
        in_specs=[pl.BlockSpec((1, C, lt), lambda b, i: (b, 0, i)),
                  pl.BlockSpec((C, 36), lambda b, i: (0, 0)),
                  pl.BlockSpec((C, 4), lambda b, i: (0, 0)),
                  pl.BlockSpec((C, lt), lambda b, i: (0, i))],
        out_specs=pl.BlockSpec((1, C, lt), lambda b, i: (b, 0, i)),
        compiler_params=pltpu.CompilerParams(
            dimension_semantics=("parallel", "parallel"),
            vmem_limit_bytes=_VMEM),
    )(h, wcols, bcols, pos)


def _res_cols(w4, b4):
    return w4.reshape(4 * 9, -1).T, b4.T


def _wt(w, K, C):
    return w.reshape(K, C).T.astype(jnp.bfloat16)


def _bcol(b):
    return b.reshape(-1, 1).astype(jnp.float32)


def kernel(x, params_key_data):
    key = jax.random.wrap_key_data(params_key_data)
    ks = jax.random.split(key, 8)
    bkeys = jax.vmap(lambda k: jax.random.split(k, 8))(ks[:4])

    w1, b1 = _init_grouped(bkeys[0, 0], (1, 2, 2), 4, 16, 4)
    wp_2, bp_2 = jax.vmap(lambda k: _init_grouped(k, (2, 4, 4), 4, 16, 4))(
        jnp.stack([bkeys[1, 0], bkeys[3, 0]]))
    wp1, bp1, wp2, bp2 = wp_2[0], bp_2[0], wp_2[1], bp_2[1]
    w2c, b2c = _init_grouped(bkeys[2, 0], (2, 4, 4), 4, 32, 4)

    w2_3, b2_3 = jax.vmap(lambda k: _init_grouped(k, (1, 3, 3), 16, 32, 16))(
        jnp.stack([bkeys[0, 1], bkeys[1, 1], bkeys[3, 1]]))
    w2_1, b2_1 = w2_3[0], b2_3[0]
    w2_p1, b2_p1 = w2_3[1], b2_3[1]
    w2_p2, b2_p2 = w2_3[2], b2_3[2]
    w2_2, b2_2 = _init_grouped(bkeys[2, 1], (1, 3, 3), 32, 64, 32)

    dwk32 = jnp.concatenate([bkeys[0, 2:6], bkeys[1, 2:6], bkeys[3, 2:6]])
    dww32, dwb32 = jax.vmap(lambda k: _init_dw(k, 3, 32))(dwk32)
    dww64, dwb64 = jax.vmap(lambda k: _init_dw(k, 3, 64))(bkeys[2, 2:6])

    pos1 = jax.random.normal(ks[4], (32, 16, 32, 32), jnp.float32) * 0.02
    pos2 = jax.random.normal(ks[5], (64, 8, 16, 16), jnp.float32) * 0.02
    pp = jax.vmap(
        lambda k: jax.random.normal(k, (32, 8, 16, 16), jnp.float32))(ks[6:8])
    pp1, pp2 = pp[0] * 0.02, pp[1] * 0.02

    B = x.shape[0]
    xb = x.astype(jnp.bfloat16)

    a1 = xb.reshape(B, 4, 16, 64, 2, 64, 2).transpose(0, 4, 6, 1, 2, 3, 5)
    a1 = a1.reshape(B, 16, 16 * 64 * 64)
    a2 = xb.reshape(B, 4, 8, 2, 32, 4, 32, 4).transpose(0, 3, 5, 7, 1, 2, 4, 6)
    a2 = a2.reshape(B, 128, 8 * 32 * 32)

    h1 = _c1_call(a1, _wt(w1, 16, 16), _bcol(b1), mt=16384)

    wcat = jnp.concatenate(
        [_wt(wp1, 128, 16), _wt(wp2, 128, 16), _wt(w2c, 128, 32)], axis=0)
    bcat = jnp.concatenate([_bcol(bp1), _bcol(bp2), _bcol(b2c)], axis=0)
    hp1, hp2, h2 = _c1b_call(a2, wcat, bcat, mt=4096)

    g1 = _c2_call(_phase_split(h1, 16, 16, 64, 64),
                  _c2_weights(w2_1, 16, 32), _bcol(b2_1),
                  mt=4096, X=32, YX=1024)
    gp1 = _c2_call(_phase_split(hp1, 16, 8, 32, 32),
                   _c2_weights(w2_p1, 16, 32), _bcol(b2_p1),
                   mt=2048, X=16, YX=256)
    gp2 = _c2_call(_phase_split(hp2, 16, 8, 32, 32),
                   _c2_weights(w2_p2, 16, 32), _bcol(b2_p2),
                   mt=2048, X=16, YX=256)
    g2 = _c2_call(_phase_split(h2, 32, 8, 32, 32),
                  _c2_weights(w2_2, 32, 64), _bcol(b2_2),
                  mt=2048, X=16, YX=256)

    o1 = _res_call(g1, *_res_cols(dww32[0:4], dwb32[0:4]), pos1.reshape(32, -1),
                   Wd=32, HW=1024, lt=8192)
    op1 = _res_call(gp1, *_res_cols(dww32[4:8], dwb32[4:8]), pp1.reshape(32, -1),
                    Wd=16, HW=256, lt=2048)
    op2 = _res_call(gp2, *_res_cols(dww32[8:12], dwb32[8:12]),
                    pp2.reshape(32, -1), Wd=16, HW=256, lt=2048)
    o2 = _res_call(g2, *_res_cols(dww64, dwb64), pos2.reshape(64, -1),
                   Wd=16, HW=256, lt=2048)

    return (o1.reshape(B, 32, 16, 32, 32),
            op1.reshape(B, 32, 8, 16, 16),
            o2.reshape(B, 64, 8, 16, 16),
            op2.reshape(B, 32, 8, 16, 16))


# speedup vs baseline: 3.8124x; 1.0316x over previous
import math

import jax
import jax.numpy as jnp
from jax.experimental import pallas as pl
from jax.experimental.pallas import tpu as pltpu

_VMEM = 48 * 1024 * 1024
_EPS = 1e-5


def _gelu(x):
    c = math.sqrt(2.0 / math.pi)
    return 0.5 * x * (1.0 + jnp.tanh(c * (x + 0.044715 * x * x * x)))


def _ln_rows(x):
    mu = jnp.mean(x, axis=0, keepdims=True)
    var = jnp.mean((x - mu) ** 2, axis=0, keepdims=True)
    return (x - mu) * jax.lax.rsqrt(var + _EPS)


def _ln2_rows(x):
    return _ln_rows(_ln_rows(x))


def _ln_seg(x, S):
    if S == 1:
        return _ln_rows(x)
    n = x.shape[0] // S
    return jnp.concatenate([_ln_rows(x[i * n:(i + 1) * n]) for i in range(S)],
                           axis=0)


def _init_grouped(key, ksize, cin, cout, groups):
    kt, kh, kw = ksize
    kkey, bkey = jax.random.split(key)
    cpg_in = cin // groups
    cpg_out = cout // groups
    scale = 1.0 / math.sqrt(kt * kh * kw * cpg_in)
    w = jax.random.normal(kkey, (kt, kh, kw, cin, cout), jnp.float32) * scale
    gi = jnp.arange(cin) // cpg_in
    go = jnp.arange(cout) // cpg_out
    w = w * (gi[:, None] == go[None, :]).astype(jnp.float32)
    b = jax.random.normal(bkey, (cout,), jnp.float32) * 0.02
    return w, b


def _init_dw(key, k, c):
    kkey, bkey = jax.random.split(key)
    w = jax.random.normal(kkey, (k, k, c), jnp.float32) * (1.0 / math.sqrt(k * k))
    b = jax.random.normal(bkey, (c,), jnp.float32) * 0.02
    return w, b


def _c1_body(a_ref, w_ref, b_ref, o_ref):
    acc = jnp.dot(w_ref[...], a_ref[0], preferred_element_type=jnp.float32)
    acc = _ln2_rows(_gelu(acc + b_ref[...]))
    o_ref[0] = acc.astype(o_ref.dtype)


def _c1_call(a, w_t, b_col, mt):
    B, K, M = a.shape
    C = w_t.shape[0]
    return pl.pallas_call(
        _c1_body,
        out_shape=jax.ShapeDtypeStruct((B, C, M), jnp.bfloat16),
        grid=(B, M // mt),
        in_specs=[pl.BlockSpec((1, K, mt), lambda b, i: (b, 0, i)),
                  pl.BlockSpec((C, K), lambda b, i: (0, 0)),
                  pl.BlockSpec((C, 1), lambda b, i: (0, 0))],
        out_specs=pl.BlockSpec((1, C, mt), lambda b, i: (b, 0, i)),
        compiler_params=pltpu.CompilerParams(
            dimension_semantics=("parallel", "parallel"),
            vmem_limit_bytes=_VMEM),
    )(a, w_t, b_col)


def _c1b_body(a_ref, w_ref, b_ref, oj_ref, o3_ref):
    acc = jnp.dot(w_ref[...], a_ref[0], preferred_element_type=jnp.float32)
    acc = _gelu(acc + b_ref[...])
    oj = jnp.concatenate([_ln2_rows(acc[0:16]), _ln2_rows(acc[16:32])], axis=0)
    oj_ref[0] = oj.astype(oj_ref.dtype)
    o3_ref[0] = _ln2_rows(acc[32:64]).astype(o3_ref.dtype)


def _c1b_call(a, w_t, b_col, mt):
    B, K, M = a.shape
    outs = [jax.ShapeDtypeStruct((B, 32, M), jnp.bfloat16),
            jax.ShapeDtypeStruct((B, 32, M), jnp.bfloat16)]
    return pl.pallas_call(
        _c1b_body,
        out_shape=outs,
        grid=(B, M // mt),
        in_specs=[pl.BlockSpec((1, K, mt), lambda b, i: (b, 0, i)),
                  pl.BlockSpec((64, K), lambda b, i: (0, 0)),
                  pl.BlockSpec((64, 1), lambda b, i: (0, 0))],
        out_specs=[pl.BlockSpec((1, 32, mt), lambda b, i: (b, 0, i)),
                   pl.BlockSpec((1, 32, mt), lambda b, i: (b, 0, i))],
        compiler_params=pltpu.CompilerParams(
            dimension_semantics=("parallel", "parallel"),
            vmem_limit_bytes=_VMEM),
    )(a, w_t, b_col)


def _c2_weights(w2, C, C2):
    w00 = w2[0, :2, :2].reshape(4 * C, C2).T
    w01a = w2[0, 0, 2].T
    w01b = w2[0, 1, 2].T
    w10 = w2[0, 2, :2].reshape(2 * C, C2).T
    w11 = w2[0, 2, 2].T
    cat = jnp.concatenate([w00, w01a, w01b, w10, w11], axis=1)
    return cat.astype(jnp.bfloat16)


def _phase_split(h, C, T, H, W):
    B = h.shape[0]
    Y, X = H // 2, W // 2
    e = h.reshape(B, C, T, Y, 2, X, 2).transpose(0, 4, 6, 1, 2, 3, 5)
    return e.reshape(B, 4 * C, T * Y * X)


def _make_c2_body(C, X, YX, S=1):
    def body(p_ref, w_ref, b_ref, o_ref):
        P = p_ref[0]
        L = P.shape[1]

        def shift(v, s):
            return jnp.concatenate([v[:, s:], v[:, :s]], axis=1)

        lane = jax.lax.broadcasted_iota(jnp.int32, (1, L), 1)
        mx = (lane % X) < (X - 1)
        my = (lane % YX) < (YX - X)
        mxy = mx & my

        zero = jnp.zeros((), jnp.bfloat16)
        s1a = jnp.where(mx, shift(P[0:C], 1), zero)
        s1b = jnp.where(mx, shift(P[2 * C:3 * C], 1), zero)
        sx = jnp.where(my, shift(P[0:2 * C], X), zero)
        sx1 = jnp.where(mxy, shift(P[0:C], X + 1), zero)
        a = jnp.concatenate([P, s1a, s1b, sx, sx1], axis=0)

        acc = jnp.dot(w_ref[...], a, preferred_element_type=jnp.float32)
        acc = _ln_seg(_gelu(acc + b_ref[...]), S)
        o_ref[0] = acc.astype(o_ref.dtype)

    return body


def _c2_call(ph, wcat, b_col, mt, X, YX, S=1):
    B, C4, M = ph.shape
    C = C4 // 4
    C2 = wcat.shape[0]
    return pl.pallas_call(
        _make_c2_body(C, X, YX, S),
        out_shape=jax.ShapeDtypeStruct((B, C2, M), jnp.bfloat16),
        grid=(B, M // mt),
        in_specs=[pl.BlockSpec((1, C4, mt), lambda b, i: (b, 0, i)),
                  pl.BlockSpec((C2, 9 * C), lambda b, i: (0, 0)),
                  pl.BlockSpec((C2, 1), lambda b, i: (0, 0))],
        out_specs=pl.BlockSpec((1, C2, mt), lambda b, i: (b, 0, i)),
        compiler_params=pltpu.CompilerParams(
            dimension_semantics=("parallel", "parallel"),
            vmem_limit_bytes=_VMEM),
    )(ph, wcat, b_col)


def _make_res_body(Wd, HW, S=1):
    taps = [(dy, dx) for dy in (-1, 0, 1) for dx in (-1, 0, 1)]

    def body(h_ref, w_ref, b_ref, pos_ref, *o_refs):
        x = h_ref[0].astype(jnp.float32)
        C, L = x.shape
        lane = jax.lax.broadcasted_iota(jnp.int32, (1, L), 1)
        py = (lane % HW) // Wd
        px = lane % Wd

        masks = []
        for dy, dx in taps:
            ok = jnp.ones((1, L), jnp.bool_)
            if dy != 0:
                ok = ok & ((py + dy >= 0) & (py + dy < HW // Wd))
            if dx != 0:
                ok = ok & ((px + dx >= 0) & (px + dx < Wd))
            masks.append(ok)

        def conv(v, wcol0):
            acc = jnp.zeros_like(v)
            for t, (dy, dx) in enumerate(taps):
                off = dy * Wd + dx
                sh = pltpu.roll(v, (-off) % L, 1) if off else v
                sh = jnp.where(masks[t], sh, 0.0)
                acc = acc + sh * w_ref[:, wcol0 + t:wcol0 + t + 1]
            return acc

        def resblock(v, j):
            a1 = _ln_seg(_gelu(conv(v, 18 * j) + b_ref[:, 2 * j:2 * j + 1]), S)
            a2 = _ln_seg(_gelu(conv(a1, 18 * j + 9)
                               + b_ref[:, 2 * j + 1:2 * j + 2]), S)
            return a2 + v

        y = _ln_seg(resblock(resblock(x, 0), 1), S)
        out = pos_ref[...] + y.astype(jnp.bfloat16).astype(jnp.float32)
        n = C // len(o_refs)
        for i, o_ref in enumerate(o_refs):
            o_ref[0] = out[i * n:(i + 1) * n]

    return body


def _res_call(h, wcols, bcols, pos, Wd, HW, lt, S=1):
    B, C, M = h.shape
    n = C // S
    outs = [jax.ShapeDtypeStruct((B, n, M), jnp.float32) for _ in range(S)]
    res = pl.pallas_call(
        _make_res_body(Wd, HW, S),
        out_shape=outs,
        grid=(B, M // lt),
        in_specs=[pl.BlockSpec((1, C, lt), lambda b, i: (b, 0, i)),
                  pl.BlockSpec((C, 36), lambda b, i: (0, 0)),
                  pl.BlockSpec((C, 4), lambda b, i: (0, 0)),
                  pl.BlockSpec((C, lt), lambda b, i: (0, i))],
        out_specs=[pl.BlockSpec((1, n, lt), lambda b, i: (b, 0, i))
                   for _ in range(S)],
        compiler_params=pltpu.CompilerParams(
            dimension_semantics=("parallel", "parallel"),
            vmem_limit_bytes=_VMEM),
    )(h, wcols, bcols, pos)
    return res


def _res_cols(w4, b4):
    return w4.reshape(4 * 9, -1).T, b4.T


def _wt(w, K, C):
    return w.reshape(K, C).T.astype(jnp.bfloat16)


def _bcol(b):
    return b.reshape(-1, 1).astype(jnp.float32)


def kernel(x, params_key_data):
    key = jax.random.wrap_key_data(params_key_data)
    ks = jax.random.split(key, 8)
    bkeys = jax.vmap(lambda k: jax.random.split(k, 8))(ks[:4])

    w1, b1 = _init_grouped(bkeys[0, 0], (1, 2, 2), 4, 16, 4)
    wp_2, bp_2 = jax.vmap(lambda k: _init_grouped(k, (2, 4, 4), 4, 16, 4))(
        jnp.stack([bkeys[1, 0], bkeys[3, 0]]))
    wp1, bp1, wp2, bp2 = wp_2[0], bp_2[0], wp_2[1], bp_2[1]
    w2c, b2c = _init_grouped(bkeys[2, 0], (2, 4, 4), 4, 32, 4)

    w2_3, b2_3 = jax.vmap(lambda k: _init_grouped(k, (1, 3, 3), 16, 32, 16))(
        jnp.stack([bkeys[0, 1], bkeys[1, 1], bkeys[3, 1]]))
    w2_1, b2_1 = w2_3[0], b2_3[0]
    w2_p1, b2_p1 = w2_3[1], b2_3[1]
    w2_p2, b2_p2 = w2_3[2], b2_3[2]
    w2_2, b2_2 = _init_grouped(bkeys[2, 1], (1, 3, 3), 32, 64, 32)

    dwk32 = jnp.concatenate([bkeys[0, 2:6], bkeys[1, 2:6], bkeys[3, 2:6]])
    dww32, dwb32 = jax.vmap(lambda k: _init_dw(k, 3, 32))(dwk32)
    dww64, dwb64 = jax.vmap(lambda k: _init_dw(k, 3, 64))(bkeys[2, 2:6])

    pos1 = jax.random.normal(ks[4], (32, 16, 32, 32), jnp.float32) * 0.02
    pos2 = jax.random.normal(ks[5], (64, 8, 16, 16), jnp.float32) * 0.02
    pp = jax.vmap(
        lambda k: jax.random.normal(k, (32, 8, 16, 16), jnp.float32))(ks[6:8])
    pp1, pp2 = pp[0] * 0.02, pp[1] * 0.02

    B = x.shape[0]
    xb = x.astype(jnp.bfloat16)

    a1 = xb.reshape(B, 4, 16, 64, 2, 64, 2).transpose(0, 4, 6, 1, 2, 3, 5)
    a1 = a1.reshape(B, 16, 16 * 64 * 64)
    a2 = xb.reshape(B, 4, 8, 2, 32, 4, 32, 4).transpose(0, 3, 5, 7, 1, 2, 4, 6)
    a2 = a2.reshape(B, 128, 8 * 32 * 32)

    h1 = _c1_call(a1, _wt(w1, 16, 16), _bcol(b1), mt=16384)

    wcat = jnp.concatenate(
        [_wt(wp1, 128, 16), _wt(wp2, 128, 16), _wt(w2c, 128, 32)], axis=0)
    bcat = jnp.concatenate([_bcol(bp1), _bcol(bp2), _bcol(b2c)], axis=0)
    hp, h2 = _c1b_call(a2, wcat, bcat, mt=4096)

    g1 = _c2_call(_phase_split(h1, 16, 16, 64, 64),
                  _c2_weights(w2_1, 16, 32), _bcol(b2_1),
                  mt=4096, X=32, YX=1024)
    w2j = jnp.zeros((1, 3, 3, 32, 64), jnp.float32)
    w2j = w2j.at[..., 0:16, 0:32].set(w2_p1).at[..., 16:32, 32:64].set(w2_p2)
    gp = _c2_call(_phase_split(hp, 32, 8, 32, 32),
                  _c2_weights(w2j, 32, 64),
                  _bcol(jnp.concatenate([b2_p1, b2_p2])),
                  mt=2048, X=16, YX=256, S=2)
    g2 = _c2_call(_phase_split(h2, 32, 8, 32, 32),
                  _c2_weights(w2_2, 32, 64), _bcol(b2_2),
                  mt=2048, X=16, YX=256)

    (o1,) = _res_call(g1, *_res_cols(dww32[0:4], dwb32[0:4]),
                      pos1.reshape(32, -1), Wd=32, HW=1024, lt=8192)
    op1, op2 = _res_call(
        gp,
        *_res_cols(jnp.concatenate([dww32[4:8], dww32[8:12]], axis=-1),
                   jnp.concatenate([dwb32[4:8], dwb32[8:12]], axis=-1)),
        jnp.concatenate([pp1.reshape(32, -1), pp2.reshape(32, -1)], axis=0),
        Wd=16, HW=256, lt=2048, S=2)
    (o2,) = _res_call(g2, *_res_cols(dww64, dwb64), pos2.reshape(64, -1),
                      Wd=16, HW=256, lt=2048)

    return (o1.reshape(B, 32, 16, 32, 32),
            op1.reshape(B, 32, 8, 16, 16),
            o2.reshape(B, 64, 8, 16, 16),
            op2.reshape(B, 32, 8, 16, 16))
